# Initial kernel scaffold; baseline (speedup 1.0000x reference)
import jax
import jax.numpy as jnp
from jax.experimental import pallas as pl


def kernel(xyz, params):
    raise NotImplementedError("write your pallas kernel here")


# speedup vs baseline: 1.0001x; 1.0001x over previous
import jax
import jax.numpy as jnp
from jax.experimental import pallas as pl


def _square_distance(src, dst):
    dist = -2.0 * jnp.einsum('bnc,bmc->bnm', src, dst)
    dist = dist + jnp.sum(src ** 2, -1)[:, :, None]
    dist = dist + jnp.sum(dst ** 2, -1)[:, None, :]
    return dist


def _index_points(points, idx):
    return jax.vmap(lambda p, i: p[i])(points, idx)


def _fps(xyz, npoint):
    B, N, _ = xyz.shape
    distance = jnp.full((B, N), 1e10, dtype=xyz.dtype)
    farthest = jnp.zeros((B,), dtype=jnp.int32)

    def step(carry, _):
        dist_c, far_c = carry
        centroid = jnp.take_along_axis(xyz, far_c[:, None, None], axis=1)
        d = jnp.sum((xyz - centroid) ** 2, -1)
        dist_c = jnp.minimum(dist_c, d)
        new_far = jnp.argmax(dist_c, -1).astype(jnp.int32)
        return (dist_c, new_far), far_c

    _, cent = jax.lax.scan(step, (distance, farthest), None, length=npoint)
    return jnp.transpose(cent)


def _query_ball(radius, nsample, xyz, new_xyz):
    B, N, _ = xyz.shape
    S = new_xyz.shape[1]
    sqrdists = _square_distance(new_xyz, xyz)
    group_idx = jnp.broadcast_to(
        jnp.arange(N, dtype=jnp.int32)[None, None, :], (B, S, N))
    group_idx = jnp.where(sqrdists > radius ** 2, N, group_idx)
    group_idx = jnp.sort(group_idx, axis=-1)[:, :, :nsample]
    group_first = jnp.broadcast_to(group_idx[:, :, :1], group_idx.shape)
    group_idx = jnp.where(group_idx == N, group_first, group_idx)
    return group_idx


def _apply_mlp(x, layers, bn_axes):
    for (W, b, g, be) in layers:
        x = x @ W.T + b
        mean = jnp.mean(x, axis=bn_axes, keepdims=True)
        var = jnp.var(x, axis=bn_axes, keepdims=True)
        x = g * (x - mean) / jnp.sqrt(var + 1e-5) + be
        x = jax.nn.relu(x)
    return x


def _set_abstraction(xyz, points, npoint, radius, nsample, layers, group_all):
    if group_all:
        new_xyz = jnp.mean(xyz, axis=1, keepdims=True)
        grouped = points[:, None, :, :]
    else:
        fps_idx = _fps(xyz, npoint)
        new_xyz = _index_points(xyz, fps_idx)
        idx = _query_ball(radius, nsample, xyz, new_xyz)
        grouped_points = _index_points(points, idx)
        grouped_xyz = _index_points(xyz, idx) - new_xyz[:, :, None, :]
        grouped = jnp.concatenate([grouped_xyz, grouped_points], axis=-1)
    x = _apply_mlp(grouped, layers, (0, 1, 2))
    new_points = jnp.max(x, axis=2)
    return new_xyz, new_points


def _feature_propagation(xyz1, xyz2, points1, points2, layers):
    M = xyz2.shape[1]
    k = min(3, M)
    dist = _square_distance(xyz1, xyz2)
    neg, idx = jax.lax.top_k(-dist, k)
    dist2 = -neg
    dist_recip = 1.0 / (dist2 + 1e-8)
    norm = jnp.sum(dist_recip, axis=2, keepdims=True)
    weight = dist_recip / norm
    gathered = _index_points(points2, idx)
    interp = jnp.sum(gathered * weight[..., None], axis=2)
    if points1 is not None:
        new_points = jnp.concatenate([points1, interp], axis=-1)
    else:
        new_points = interp
    return _apply_mlp(new_points, layers, (0, 1))


def kernel(xyz, params):
    l0_xyz = xyz[:, :, :3]
    l0_points = xyz
    l1_xyz, l1_points = _set_abstraction(
        l0_xyz, l0_points, 1024, 0.2, 32, params['sa1'], False)
    l2_xyz, l2_points = _set_abstraction(
        l1_xyz, l1_points, 256, 0.4, 32, params['sa2'], False)
    l3_xyz, l3_points = _set_abstraction(
        l2_xyz, l2_points, 64, 0.8, 32, params['sa3'], False)
    l4_xyz, l4_points = _set_abstraction(
        l3_xyz, l3_points, None, None, None, params['sa4'], True)
    l3_points = _feature_propagation(
        l3_xyz, l4_xyz, l3_points, l4_points, params['fp4'])
    l2_points = _feature_propagation(
        l2_xyz, l3_xyz, l2_points, l3_points, params['fp3'])
    l1_points = _feature_propagation(
        l1_xyz, l2_xyz, l1_points, l2_points, params['fp2'])
    l0_out = _feature_propagation(
        l0_xyz, l1_xyz, l0_points, l1_points, params['fp1'])
    x = _apply_mlp(l0_out, params['head'], (0, 1))
    W2, b2 = params['conv2']
    x = x @ W2.T + b2
    return x


# speedup vs baseline: 1.6092x; 1.6091x over previous
import functools

import jax
import jax.numpy as jnp
from jax.experimental import pallas as pl

_NL = 128


def _fps_body(npoint, N, x_ref, out_ref):
    B = x_ref.shape[0]
    NS = x_ref.shape[2]
    x = x_ref[:, 0, :, :]
    y = x_ref[:, 1, :, :]
    z = x_ref[:, 2, :, :]
    iota_flat = (jax.lax.broadcasted_iota(jnp.int32, (B, NS, _NL), 1) * _NL
                 + jax.lax.broadcasted_iota(jnp.int32, (B, NS, _NL), 2))
    iota_np = jax.lax.broadcasted_iota(jnp.int32, (B, npoint), 1)

    dist0 = jnp.full((B, NS, _NL), 1e10, dtype=jnp.float32)
    far0 = jnp.zeros((B, 1, 1), dtype=jnp.int32)
    acc0 = jnp.zeros((B, npoint), dtype=jnp.int32)

    def step(j, carry):
        dist, far, acc = carry
        acc = jnp.where(iota_np == j, far[:, :, 0], acc)
        sel = iota_flat == far
        cx = jnp.sum(jnp.where(sel, x, 0.0), axis=(1, 2), keepdims=True)
        cy = jnp.sum(jnp.where(sel, y, 0.0), axis=(1, 2), keepdims=True)
        cz = jnp.sum(jnp.where(sel, z, 0.0), axis=(1, 2), keepdims=True)
        d = (x - cx) ** 2 + (y - cy) ** 2 + (z - cz) ** 2
        dist = jnp.minimum(dist, d)
        m = jnp.max(dist, axis=(1, 2), keepdims=True)
        far2 = jnp.min(jnp.where(dist == m, iota_flat, N), axis=(1, 2),
                       keepdims=True).astype(jnp.int32)
        return dist, far2, acc

    _, _, acc = jax.lax.fori_loop(0, npoint, step, (dist0, far0, acc0))
    out_ref[:, :] = acc


def _fps_pallas(xyz, npoint):
    B, N, _ = xyz.shape
    NS = N // _NL
    xr = jnp.transpose(xyz, (0, 2, 1)).reshape(B, 3, NS, _NL)
    return pl.pallas_call(
        functools.partial(_fps_body, npoint, N),
        out_shape=jax.ShapeDtypeStruct((B, npoint), jnp.int32),
    )(xr)


def _square_distance(src, dst):
    dist = -2.0 * jnp.einsum('bnc,bmc->bnm', src, dst)
    dist = dist + jnp.sum(src ** 2, -1)[:, :, None]
    dist = dist + jnp.sum(dst ** 2, -1)[:, None, :]
    return dist


def _index_points(points, idx):
    return jax.vmap(lambda p, i: p[i])(points, idx)


def _fps(xyz, npoint):
    B, N, _ = xyz.shape
    distance = jnp.full((B, N), 1e10, dtype=xyz.dtype)
    farthest = jnp.zeros((B,), dtype=jnp.int32)

    def step(carry, _):
        dist_c, far_c = carry
        centroid = jnp.take_along_axis(xyz, far_c[:, None, None], axis=1)
        d = jnp.sum((xyz - centroid) ** 2, -1)
        dist_c = jnp.minimum(dist_c, d)
        new_far = jnp.argmax(dist_c, -1).astype(jnp.int32)
        return (dist_c, new_far), far_c

    _, cent = jax.lax.scan(step, (distance, farthest), None, length=npoint)
    return jnp.transpose(cent)


def _query_ball(radius, nsample, xyz, new_xyz):
    B, N, _ = xyz.shape
    S = new_xyz.shape[1]
    sqrdists = _square_distance(new_xyz, xyz)
    group_idx = jnp.broadcast_to(
        jnp.arange(N, dtype=jnp.int32)[None, None, :], (B, S, N))
    group_idx = jnp.where(sqrdists > radius ** 2, N, group_idx)
    group_idx = jnp.sort(group_idx, axis=-1)[:, :, :nsample]
    group_first = jnp.broadcast_to(group_idx[:, :, :1], group_idx.shape)
    group_idx = jnp.where(group_idx == N, group_first, group_idx)
    return group_idx


def _apply_mlp(x, layers, bn_axes):
    for (W, b, g, be) in layers:
        x = x @ W.T + b
        mean = jnp.mean(x, axis=bn_axes, keepdims=True)
        var = jnp.var(x, axis=bn_axes, keepdims=True)
        x = g * (x - mean) / jnp.sqrt(var + 1e-5) + be
        x = jax.nn.relu(x)
    return x


def _set_abstraction(xyz, points, npoint, radius, nsample, layers, group_all):
    if group_all:
        new_xyz = jnp.mean(xyz, axis=1, keepdims=True)
        grouped = points[:, None, :, :]
    else:
        fps_idx = _fps_pallas(xyz, npoint)
        new_xyz = _index_points(xyz, fps_idx)
        idx = _query_ball(radius, nsample, xyz, new_xyz)
        grouped_points = _index_points(points, idx)
        grouped_xyz = _index_points(xyz, idx) - new_xyz[:, :, None, :]
        grouped = jnp.concatenate([grouped_xyz, grouped_points], axis=-1)
    x = _apply_mlp(grouped, layers, (0, 1, 2))
    new_points = jnp.max(x, axis=2)
    return new_xyz, new_points


def _feature_propagation(xyz1, xyz2, points1, points2, layers):
    M = xyz2.shape[1]
    k = min(3, M)
    dist = _square_distance(xyz1, xyz2)
    neg, idx = jax.lax.top_k(-dist, k)
    dist2 = -neg
    dist_recip = 1.0 / (dist2 + 1e-8)
    norm = jnp.sum(dist_recip, axis=2, keepdims=True)
    weight = dist_recip / norm
    gathered = _index_points(points2, idx)
    interp = jnp.sum(gathered * weight[..., None], axis=2)
    if points1 is not None:
        new_points = jnp.concatenate([points1, interp], axis=-1)
    else:
        new_points = interp
    return _apply_mlp(new_points, layers, (0, 1))


def kernel(xyz, params):
    l0_xyz = xyz[:, :, :3]
    l0_points = xyz
    l1_xyz, l1_points = _set_abstraction(
        l0_xyz, l0_points, 1024, 0.2, 32, params['sa1'], False)
    l2_xyz, l2_points = _set_abstraction(
        l1_xyz, l1_points, 256, 0.4, 32, params['sa2'], False)
    l3_xyz, l3_points = _set_abstraction(
        l2_xyz, l2_points, 64, 0.8, 32, params['sa3'], False)
    l4_xyz, l4_points = _set_abstraction(
        l3_xyz, l3_points, None, None, None, params['sa4'], True)
    l3_points = _feature_propagation(
        l3_xyz, l4_xyz, l3_points, l4_points, params['fp4'])
    l2_points = _feature_propagation(
        l2_xyz, l3_xyz, l2_points, l3_points, params['fp3'])
    l1_points = _feature_propagation(
        l1_xyz, l2_xyz, l1_points, l2_points, params['fp2'])
    l0_out = _feature_propagation(
        l0_xyz, l1_xyz, l0_points, l1_points, params['fp1'])
    x = _apply_mlp(l0_out, params['head'], (0, 1))
    W2, b2 = params['conv2']
    x = x @ W2.T + b2
    return x


# speedup vs baseline: 1.9337x; 1.2016x over previous
import functools

import jax
import jax.numpy as jnp
from jax.experimental import pallas as pl

_NL = 128


def _fps_body(npoint, N, x_ref, out_ref):
    B = x_ref.shape[0]
    NS = x_ref.shape[2]
    x = x_ref[:, 0, :, :]
    y = x_ref[:, 1, :, :]
    z = x_ref[:, 2, :, :]
    iota_flat = (jax.lax.broadcasted_iota(jnp.int32, (B, NS, _NL), 1) * _NL
                 + jax.lax.broadcasted_iota(jnp.int32, (B, NS, _NL), 2))
    iota_np = jax.lax.broadcasted_iota(jnp.int32, (B, npoint), 1)

    dist0 = jnp.full((B, NS, _NL), 1e10, dtype=jnp.float32)
    far0 = jnp.zeros((B, 1, 1), dtype=jnp.int32)
    acc0 = jnp.zeros((B, npoint), dtype=jnp.int32)

    def step(j, carry):
        dist, far, acc = carry
        acc = jnp.where(iota_np == j, far[:, :, 0], acc)
        sel = iota_flat == far
        cx = jnp.sum(jnp.where(sel, x, 0.0), axis=(1, 2), keepdims=True)
        cy = jnp.sum(jnp.where(sel, y, 0.0), axis=(1, 2), keepdims=True)
        cz = jnp.sum(jnp.where(sel, z, 0.0), axis=(1, 2), keepdims=True)
        d = (x - cx) ** 2 + (y - cy) ** 2 + (z - cz) ** 2
        dist = jnp.minimum(dist, d)
        m = jnp.max(dist, axis=(1, 2), keepdims=True)
        far2 = jnp.min(jnp.where(dist == m, iota_flat, N), axis=(1, 2),
                       keepdims=True).astype(jnp.int32)
        return dist, far2, acc

    _, _, acc = jax.lax.fori_loop(0, npoint, step, (dist0, far0, acc0))
    out_ref[:, :] = acc


def _fps_pallas(xyz, npoint):
    B, N, _ = xyz.shape
    NS = N // _NL
    xr = jnp.transpose(xyz, (0, 2, 1)).reshape(B, 3, NS, _NL)
    return pl.pallas_call(
        functools.partial(_fps_body, npoint, N),
        out_shape=jax.ShapeDtypeStruct((B, npoint), jnp.int32),
    )(xr)


def _square_distance(src, dst):
    dist = -2.0 * jnp.einsum('bnc,bmc->bnm', src, dst)
    dist = dist + jnp.sum(src ** 2, -1)[:, :, None]
    dist = dist + jnp.sum(dst ** 2, -1)[:, None, :]
    return dist


def _index_points(points, idx):
    return jax.vmap(lambda p, i: p[i])(points, idx)


def _fps(xyz, npoint):
    B, N, _ = xyz.shape
    distance = jnp.full((B, N), 1e10, dtype=xyz.dtype)
    farthest = jnp.zeros((B,), dtype=jnp.int32)

    def step(carry, _):
        dist_c, far_c = carry
        centroid = jnp.take_along_axis(xyz, far_c[:, None, None], axis=1)
        d = jnp.sum((xyz - centroid) ** 2, -1)
        dist_c = jnp.minimum(dist_c, d)
        new_far = jnp.argmax(dist_c, -1).astype(jnp.int32)
        return (dist_c, new_far), far_c

    _, cent = jax.lax.scan(step, (distance, farthest), None, length=npoint)
    return jnp.transpose(cent)


def _bq_body(r2, N, nsample, q_ref, x_ref, out_ref):
    q = q_ref[0]
    xT = x_ref[0]
    St = q.shape[0]
    d = -2.0 * jax.lax.dot(q, xT, precision=jax.lax.Precision.HIGHEST,
                           preferred_element_type=jnp.float32)
    d = d + jnp.sum(q * q, axis=-1, keepdims=True)
    d = d + jnp.sum(xT * xT, axis=0, keepdims=True)
    iota_n = jax.lax.broadcasted_iota(jnp.int32, (St, N), 1)
    iota_k = jax.lax.broadcasted_iota(jnp.int32, (St, nsample), 1)
    keys0 = jnp.where(d <= r2, iota_n, N)
    acc0 = jnp.zeros((St, nsample), dtype=jnp.int32)

    def step(j, carry):
        keys, acc = carry
        m = jnp.min(keys, axis=-1, keepdims=True)
        acc = jnp.where(iota_k == j, m, acc)
        keys = jnp.where(keys == m, N, keys)
        return keys, acc

    _, acc = jax.lax.fori_loop(0, nsample, step, (keys0, acc0))
    acc = jnp.where(acc == N, acc[:, 0:1], acc)
    out_ref[0] = acc


def _query_ball(radius, nsample, xyz, new_xyz):
    B, N, _ = xyz.shape
    S = new_xyz.shape[1]
    St = min(S, 128)
    xT = jnp.transpose(xyz, (0, 2, 1))
    grid = (B, S // St)
    return pl.pallas_call(
        functools.partial(_bq_body, radius * radius, N, nsample),
        grid=grid,
        in_specs=[
            pl.BlockSpec((1, St, 3), lambda b, s: (b, s, 0)),
            pl.BlockSpec((1, 3, N), lambda b, s: (b, 0, 0)),
        ],
        out_specs=pl.BlockSpec((1, St, nsample), lambda b, s: (b, s, 0)),
        out_shape=jax.ShapeDtypeStruct((B, S, nsample), jnp.int32),
    )(new_xyz, xT)


def _apply_mlp(x, layers, bn_axes):
    for (W, b, g, be) in layers:
        x = x @ W.T + b
        mean = jnp.mean(x, axis=bn_axes, keepdims=True)
        var = jnp.var(x, axis=bn_axes, keepdims=True)
        x = g * (x - mean) / jnp.sqrt(var + 1e-5) + be
        x = jax.nn.relu(x)
    return x


def _set_abstraction(xyz, points, npoint, radius, nsample, layers, group_all):
    if group_all:
        new_xyz = jnp.mean(xyz, axis=1, keepdims=True)
        grouped = points[:, None, :, :]
    else:
        fps_idx = _fps_pallas(xyz, npoint)
        new_xyz = _index_points(xyz, fps_idx)
        idx = _query_ball(radius, nsample, xyz, new_xyz)
        grouped_points = _index_points(points, idx)
        grouped_xyz = _index_points(xyz, idx) - new_xyz[:, :, None, :]
        grouped = jnp.concatenate([grouped_xyz, grouped_points], axis=-1)
    x = _apply_mlp(grouped, layers, (0, 1, 2))
    new_points = jnp.max(x, axis=2)
    return new_xyz, new_points


def _feature_propagation(xyz1, xyz2, points1, points2, layers):
    M = xyz2.shape[1]
    k = min(3, M)
    dist = _square_distance(xyz1, xyz2)
    neg, idx = jax.lax.top_k(-dist, k)
    dist2 = -neg
    dist_recip = 1.0 / (dist2 + 1e-8)
    norm = jnp.sum(dist_recip, axis=2, keepdims=True)
    weight = dist_recip / norm
    gathered = _index_points(points2, idx)
    interp = jnp.sum(gathered * weight[..., None], axis=2)
    if points1 is not None:
        new_points = jnp.concatenate([points1, interp], axis=-1)
    else:
        new_points = interp
    return _apply_mlp(new_points, layers, (0, 1))


def kernel(xyz, params):
    l0_xyz = xyz[:, :, :3]
    l0_points = xyz
    l1_xyz, l1_points = _set_abstraction(
        l0_xyz, l0_points, 1024, 0.2, 32, params['sa1'], False)
    l2_xyz, l2_points = _set_abstraction(
        l1_xyz, l1_points, 256, 0.4, 32, params['sa2'], False)
    l3_xyz, l3_points = _set_abstraction(
        l2_xyz, l2_points, 64, 0.8, 32, params['sa3'], False)
    l4_xyz, l4_points = _set_abstraction(
        l3_xyz, l3_points, None, None, None, params['sa4'], True)
    l3_points = _feature_propagation(
        l3_xyz, l4_xyz, l3_points, l4_points, params['fp4'])
    l2_points = _feature_propagation(
        l2_xyz, l3_xyz, l2_points, l3_points, params['fp3'])
    l1_points = _feature_propagation(
        l1_xyz, l2_xyz, l1_points, l2_points, params['fp2'])
    l0_out = _feature_propagation(
        l0_xyz, l1_xyz, l0_points, l1_points, params['fp1'])
    x = _apply_mlp(l0_out, params['head'], (0, 1))
    W2, b2 = params['conv2']
    x = x @ W2.T + b2
    return x


# speedup vs baseline: 1.9400x; 1.0033x over previous
import functools

import jax
import jax.numpy as jnp
from jax.experimental import pallas as pl

_NL = 128


def _fps_body(npoint, N, x_ref, out_ref):
    B = x_ref.shape[0]
    NS = x_ref.shape[2]
    x = x_ref[:, 0, :, :]
    y = x_ref[:, 1, :, :]
    z = x_ref[:, 2, :, :]
    iota_flat = (jax.lax.broadcasted_iota(jnp.int32, (B, NS, _NL), 1) * _NL
                 + jax.lax.broadcasted_iota(jnp.int32, (B, NS, _NL), 2))
    iota_np = jax.lax.broadcasted_iota(jnp.int32, (B, npoint), 1)

    dist0 = jnp.full((B, NS, _NL), 1e10, dtype=jnp.float32)
    far0 = jnp.zeros((B, 1, 1), dtype=jnp.int32)
    acc0 = jnp.zeros((B, npoint), dtype=jnp.int32)

    def step(j, carry):
        dist, far, acc = carry
        acc = jnp.where(iota_np == j, far[:, :, 0], acc)
        sel = iota_flat == far
        cx = jnp.sum(jnp.where(sel, x, 0.0), axis=(1, 2), keepdims=True)
        cy = jnp.sum(jnp.where(sel, y, 0.0), axis=(1, 2), keepdims=True)
        cz = jnp.sum(jnp.where(sel, z, 0.0), axis=(1, 2), keepdims=True)
        d = (x - cx) ** 2 + (y - cy) ** 2 + (z - cz) ** 2
        dist = jnp.minimum(dist, d)
        m = jnp.max(dist, axis=(1, 2), keepdims=True)
        far2 = jnp.min(jnp.where(dist == m, iota_flat, N), axis=(1, 2),
                       keepdims=True).astype(jnp.int32)
        return dist, far2, acc

    _, _, acc = jax.lax.fori_loop(0, npoint, step, (dist0, far0, acc0))
    out_ref[:, :] = acc


def _fps_pallas(xyz, npoint):
    B, N, _ = xyz.shape
    NS = N // _NL
    xr = jnp.transpose(xyz, (0, 2, 1)).reshape(B, 3, NS, _NL)
    return pl.pallas_call(
        functools.partial(_fps_body, npoint, N),
        out_shape=jax.ShapeDtypeStruct((B, npoint), jnp.int32),
    )(xr)


def _square_distance(src, dst):
    dist = -2.0 * jnp.einsum('bnc,bmc->bnm', src, dst)
    dist = dist + jnp.sum(src ** 2, -1)[:, :, None]
    dist = dist + jnp.sum(dst ** 2, -1)[:, None, :]
    return dist


def _index_points(points, idx):
    return jax.vmap(lambda p, i: p[i])(points, idx)


def _fps(xyz, npoint):
    B, N, _ = xyz.shape
    distance = jnp.full((B, N), 1e10, dtype=xyz.dtype)
    farthest = jnp.zeros((B,), dtype=jnp.int32)

    def step(carry, _):
        dist_c, far_c = carry
        centroid = jnp.take_along_axis(xyz, far_c[:, None, None], axis=1)
        d = jnp.sum((xyz - centroid) ** 2, -1)
        dist_c = jnp.minimum(dist_c, d)
        new_far = jnp.argmax(dist_c, -1).astype(jnp.int32)
        return (dist_c, new_far), far_c

    _, cent = jax.lax.scan(step, (distance, farthest), None, length=npoint)
    return jnp.transpose(cent)


def _bq_body(r2, N, nsample, q_ref, x_ref, out_ref):
    q = q_ref[0]
    xT = x_ref[0]
    St = q.shape[0]
    d = -2.0 * jax.lax.dot(q, xT, preferred_element_type=jnp.float32)
    d = d + jnp.sum(q * q, axis=-1, keepdims=True)
    d = d + jnp.sum(xT * xT, axis=0, keepdims=True)
    iota_n = jax.lax.broadcasted_iota(jnp.int32, (St, N), 1)
    iota_k = jax.lax.broadcasted_iota(jnp.int32, (St, nsample), 1)
    keys0 = jnp.where(d <= r2, iota_n, N)
    acc0 = jnp.zeros((St, nsample), dtype=jnp.int32)

    def step(j, carry):
        keys, acc = carry
        m = jnp.min(keys, axis=-1, keepdims=True)
        acc = jnp.where(iota_k == j, m, acc)
        keys = jnp.where(keys == m, N, keys)
        return keys, acc

    _, acc = jax.lax.fori_loop(0, nsample, step, (keys0, acc0))
    acc = jnp.where(acc == N, acc[:, 0:1], acc)
    out_ref[0] = acc


def _query_ball(radius, nsample, xyz, new_xyz):
    B, N, _ = xyz.shape
    S = new_xyz.shape[1]
    St = min(S, 128)
    xT = jnp.transpose(xyz, (0, 2, 1))
    grid = (B, S // St)
    return pl.pallas_call(
        functools.partial(_bq_body, radius * radius, N, nsample),
        grid=grid,
        in_specs=[
            pl.BlockSpec((1, St, 3), lambda b, s: (b, s, 0)),
            pl.BlockSpec((1, 3, N), lambda b, s: (b, 0, 0)),
        ],
        out_specs=pl.BlockSpec((1, St, nsample), lambda b, s: (b, s, 0)),
        out_shape=jax.ShapeDtypeStruct((B, S, nsample), jnp.int32),
    )(new_xyz, xT)


def _apply_mlp(x, layers, bn_axes):
    for (W, b, g, be) in layers:
        x = x @ W.T + b
        mean = jnp.mean(x, axis=bn_axes, keepdims=True)
        var = jnp.var(x, axis=bn_axes, keepdims=True)
        x = g * (x - mean) / jnp.sqrt(var + 1e-5) + be
        x = jax.nn.relu(x)
    return x


def _set_abstraction(xyz, points, npoint, radius, nsample, layers, group_all):
    if group_all:
        new_xyz = jnp.mean(xyz, axis=1, keepdims=True)
        grouped = points[:, None, :, :]
    else:
        fps_idx = _fps_pallas(xyz, npoint)
        new_xyz = _index_points(xyz, fps_idx)
        idx = _query_ball(radius, nsample, xyz, new_xyz)
        grouped_points = _index_points(points, idx)
        grouped_xyz = _index_points(xyz, idx) - new_xyz[:, :, None, :]
        grouped = jnp.concatenate([grouped_xyz, grouped_points], axis=-1)
    x = _apply_mlp(grouped, layers, (0, 1, 2))
    new_points = jnp.max(x, axis=2)
    return new_xyz, new_points


def _feature_propagation(xyz1, xyz2, points1, points2, layers):
    M = xyz2.shape[1]
    k = min(3, M)
    dist = _square_distance(xyz1, xyz2)
    neg, idx = jax.lax.top_k(-dist, k)
    dist2 = -neg
    dist_recip = 1.0 / (dist2 + 1e-8)
    norm = jnp.sum(dist_recip, axis=2, keepdims=True)
    weight = dist_recip / norm
    gathered = _index_points(points2, idx)
    interp = jnp.sum(gathered * weight[..., None], axis=2)
    if points1 is not None:
        new_points = jnp.concatenate([points1, interp], axis=-1)
    else:
        new_points = interp
    return _apply_mlp(new_points, layers, (0, 1))


def kernel(xyz, params):
    l0_xyz = xyz[:, :, :3]
    l0_points = xyz
    l1_xyz, l1_points = _set_abstraction(
        l0_xyz, l0_points, 1024, 0.2, 32, params['sa1'], False)
    l2_xyz, l2_points = _set_abstraction(
        l1_xyz, l1_points, 256, 0.4, 32, params['sa2'], False)
    l3_xyz, l3_points = _set_abstraction(
        l2_xyz, l2_points, 64, 0.8, 32, params['sa3'], False)
    l4_xyz, l4_points = _set_abstraction(
        l3_xyz, l3_points, None, None, None, params['sa4'], True)
    l3_points = _feature_propagation(
        l3_xyz, l4_xyz, l3_points, l4_points, params['fp4'])
    l2_points = _feature_propagation(
        l2_xyz, l3_xyz, l2_points, l3_points, params['fp3'])
    l1_points = _feature_propagation(
        l1_xyz, l2_xyz, l1_points, l2_points, params['fp2'])
    l0_out = _feature_propagation(
        l0_xyz, l1_xyz, l0_points, l1_points, params['fp1'])
    x = _apply_mlp(l0_out, params['head'], (0, 1))
    W2, b2 = params['conv2']
    x = x @ W2.T + b2
    return x


# speedup vs baseline: 2.1133x; 1.0893x over previous
import functools

import jax
import jax.numpy as jnp
from jax.experimental import pallas as pl

_NL = 128


def _fps_body(npoint, N, x_ref, out_ref):
    B = x_ref.shape[0]
    NS = x_ref.shape[2]
    x = x_ref[:, 0, :, :]
    y = x_ref[:, 1, :, :]
    z = x_ref[:, 2, :, :]
    iota_flat = (jax.lax.broadcasted_iota(jnp.int32, (B, NS, _NL), 1) * _NL
                 + jax.lax.broadcasted_iota(jnp.int32, (B, NS, _NL), 2))
    iota_np = jax.lax.broadcasted_iota(jnp.int32, (B, npoint), 1)

    dist0 = jnp.full((B, NS, _NL), 1e10, dtype=jnp.float32)
    far0 = jnp.zeros((B, 1, 1), dtype=jnp.int32)
    acc0 = jnp.zeros((B, npoint), dtype=jnp.int32)

    def step(j, carry):
        dist, far, acc = carry
        acc = jnp.where(iota_np == j, far[:, :, 0], acc)
        sel = iota_flat == far
        cx = jnp.sum(jnp.where(sel, x, 0.0), axis=(1, 2), keepdims=True)
        cy = jnp.sum(jnp.where(sel, y, 0.0), axis=(1, 2), keepdims=True)
        cz = jnp.sum(jnp.where(sel, z, 0.0), axis=(1, 2), keepdims=True)
        d = (x - cx) ** 2 + (y - cy) ** 2 + (z - cz) ** 2
        dist = jnp.minimum(dist, d)
        m = jnp.max(dist, axis=(1, 2), keepdims=True)
        far2 = jnp.min(jnp.where(dist == m, iota_flat, N), axis=(1, 2),
                       keepdims=True).astype(jnp.int32)
        return dist, far2, acc

    _, _, acc = jax.lax.fori_loop(0, npoint, step, (dist0, far0, acc0))
    out_ref[:, :] = acc


def _fps_pallas(xyz, npoint):
    B, N, _ = xyz.shape
    NS = N // _NL
    xr = jnp.transpose(xyz, (0, 2, 1)).reshape(B, 3, NS, _NL)
    return pl.pallas_call(
        functools.partial(_fps_body, npoint, N),
        out_shape=jax.ShapeDtypeStruct((B, npoint), jnp.int32),
    )(xr)


def _square_distance(src, dst):
    dist = -2.0 * jnp.einsum('bnc,bmc->bnm', src, dst)
    dist = dist + jnp.sum(src ** 2, -1)[:, :, None]
    dist = dist + jnp.sum(dst ** 2, -1)[:, None, :]
    return dist


def _index_points(points, idx):
    return jax.vmap(lambda p, i: p[i])(points, idx)


def _fps(xyz, npoint):
    B, N, _ = xyz.shape
    distance = jnp.full((B, N), 1e10, dtype=xyz.dtype)
    farthest = jnp.zeros((B,), dtype=jnp.int32)

    def step(carry, _):
        dist_c, far_c = carry
        centroid = jnp.take_along_axis(xyz, far_c[:, None, None], axis=1)
        d = jnp.sum((xyz - centroid) ** 2, -1)
        dist_c = jnp.minimum(dist_c, d)
        new_far = jnp.argmax(dist_c, -1).astype(jnp.int32)
        return (dist_c, new_far), far_c

    _, cent = jax.lax.scan(step, (distance, farthest), None, length=npoint)
    return jnp.transpose(cent)


def _bq_body(N, nsample, keys_ref, out_ref):
    keys0 = keys_ref[0]
    St = keys0.shape[0]
    iota_k = jax.lax.broadcasted_iota(jnp.int32, (St, nsample), 1)
    acc0 = jnp.zeros((St, nsample), dtype=jnp.int32)

    def step(j, carry):
        keys, acc = carry
        m = jnp.min(keys, axis=-1, keepdims=True)
        acc = jnp.where(iota_k == j, m, acc)
        keys = jnp.where(keys == m, N, keys)
        return keys, acc

    _, acc = jax.lax.fori_loop(0, nsample, step, (keys0, acc0))
    acc = jnp.where(acc == N, acc[:, 0:1], acc)
    out_ref[0] = acc


def _query_ball(radius, nsample, xyz, new_xyz):
    B, N, _ = xyz.shape
    S = new_xyz.shape[1]
    St = min(S, 128)
    sqrdists = _square_distance(new_xyz, xyz)
    iota_n = jnp.arange(N, dtype=jnp.int32)
    keys = jnp.where(sqrdists > radius ** 2, N, iota_n[None, None, :])
    grid = (B, S // St)
    return pl.pallas_call(
        functools.partial(_bq_body, N, nsample),
        grid=grid,
        in_specs=[
            pl.BlockSpec((1, St, N), lambda b, s: (b, s, 0)),
        ],
        out_specs=pl.BlockSpec((1, St, nsample), lambda b, s: (b, s, 0)),
        out_shape=jax.ShapeDtypeStruct((B, S, nsample), jnp.int32),
    )(keys)


def _apply_mlp(x, layers, bn_axes):
    for (W, b, g, be) in layers:
        x = x @ W.T + b
        mean = jnp.mean(x, axis=bn_axes, keepdims=True)
        var = jnp.var(x, axis=bn_axes, keepdims=True)
        x = g * (x - mean) / jnp.sqrt(var + 1e-5) + be
        x = jax.nn.relu(x)
    return x


def _set_abstraction(xyz, points, npoint, radius, nsample, layers, group_all):
    if group_all:
        new_xyz = jnp.mean(xyz, axis=1, keepdims=True)
        grouped = points[:, None, :, :]
    else:
        fps_idx = _fps_pallas(xyz, npoint)
        new_xyz = _index_points(xyz, fps_idx)
        idx = _query_ball(radius, nsample, xyz, new_xyz)
        grouped_points = _index_points(points, idx)
        grouped_xyz = _index_points(xyz, idx) - new_xyz[:, :, None, :]
        grouped = jnp.concatenate([grouped_xyz, grouped_points], axis=-1)
    x = _apply_mlp(grouped, layers, (0, 1, 2))
    new_points = jnp.max(x, axis=2)
    return new_xyz, new_points


def _fp_body(n2, d2_ref, idx_ref, p2_ref, out_ref):
    d2 = d2_ref[0]
    idx = idx_ref[0]
    p2 = p2_ref[0]
    St = d2.shape[0]
    iota_n = jax.lax.broadcasted_iota(jnp.int32, (St, n2), 1)
    r = [1.0 / (d2[:, j:j + 1] + 1e-8) for j in range(3)]
    norm = r[0] + r[1] + r[2]
    a = jnp.concatenate(
        [(iota_n == idx[:, j:j + 1]).astype(jnp.float32) for j in range(3)],
        axis=0)
    g = jax.lax.dot(a, p2, precision=jax.lax.Precision.HIGHEST,
                    preferred_element_type=jnp.float32)
    out_ref[0] = ((g[0:St] * (r[0] / norm) + g[St:2 * St] * (r[1] / norm))
                  + g[2 * St:3 * St] * (r[2] / norm))


def _fp_interp_pallas(dist2, idx, points2, interpret=False):
    B, n1, _ = dist2.shape
    n2, C2 = points2.shape[-2], points2.shape[-1]
    St = min(n1, 128)
    grid = (B, n1 // St)
    return pl.pallas_call(
        functools.partial(_fp_body, n2),
        grid=grid,
        in_specs=[
            pl.BlockSpec((1, St, 3), lambda b, s: (b, s, 0)),
            pl.BlockSpec((1, St, 3), lambda b, s: (b, s, 0)),
            pl.BlockSpec((1, n2, C2), lambda b, s: (b, 0, 0)),
        ],
        out_specs=pl.BlockSpec((1, St, C2), lambda b, s: (b, s, 0)),
        out_shape=jax.ShapeDtypeStruct((B, n1, C2), jnp.float32),
        interpret=interpret,
    )(dist2, idx, points2)


def _feature_propagation(xyz1, xyz2, points1, points2, layers):
    M = xyz2.shape[1]
    if M < 3:
        interp = jnp.broadcast_to(
            points2[:, :1, :], (points2.shape[0], xyz1.shape[1],
                                points2.shape[2]))
    else:
        dist = _square_distance(xyz1, xyz2)
        neg, idx = jax.lax.top_k(-dist, 3)
        interp = _fp_interp_pallas(-neg, idx, points2)
    if points1 is not None:
        new_points = jnp.concatenate([points1, interp], axis=-1)
    else:
        new_points = interp
    return _apply_mlp(new_points, layers, (0, 1))


def kernel(xyz, params):
    l0_xyz = xyz[:, :, :3]
    l0_points = xyz
    l1_xyz, l1_points = _set_abstraction(
        l0_xyz, l0_points, 1024, 0.2, 32, params['sa1'], False)
    l2_xyz, l2_points = _set_abstraction(
        l1_xyz, l1_points, 256, 0.4, 32, params['sa2'], False)
    l3_xyz, l3_points = _set_abstraction(
        l2_xyz, l2_points, 64, 0.8, 32, params['sa3'], False)
    l4_xyz, l4_points = _set_abstraction(
        l3_xyz, l3_points, None, None, None, params['sa4'], True)
    l3_points = _feature_propagation(
        l3_xyz, l4_xyz, l3_points, l4_points, params['fp4'])
    l2_points = _feature_propagation(
        l2_xyz, l3_xyz, l2_points, l3_points, params['fp3'])
    l1_points = _feature_propagation(
        l1_xyz, l2_xyz, l1_points, l2_points, params['fp2'])
    l0_out = _feature_propagation(
        l0_xyz, l1_xyz, l0_points, l1_points, params['fp1'])
    x = _apply_mlp(l0_out, params['head'], (0, 1))
    W2, b2 = params['conv2']
    x = x @ W2.T + b2
    return x
